# baseline (speedup 1.0000x reference)
import jax
import jax.numpy as jnp
from jax.experimental import pallas as pl
from jax.experimental.pallas import tpu as pltpu


def _xattn_kernel(x_ref, m_ref, e_ref, tail_ref, wxo_ref, weo_ref, ws_ref,
                  out_ref, carry_ref):
    j = pl.program_id(1)
    x = x_ref[0, 0]
    e = e_ref[0, 0]
    wxo = wxo_ref[...]
    weo = weo_ref[...]

    vexp = (jnp.dot(x, wxo, preferred_element_type=jnp.float32)
            + jnp.dot(e, weo, preferred_element_type=jnp.float32))

    tx = tail_ref[0, 0:1, :]
    te = tail_ref[0, 2:3, :]
    tail_vexp = (jnp.dot(tx, wxo, preferred_element_type=jnp.float32)
                 + jnp.dot(te, weo, preferred_element_type=jnp.float32))
    vprev = jnp.where(j == 0, tail_vexp, carry_ref[0:1, 0:3])
    carry_ref[0:1, 0:3] = vexp[-1:, :]

    rolled = pltpu.roll(vexp, 1, 0)
    row = jax.lax.broadcasted_iota(jnp.int32, rolled.shape, 0)
    w = jnp.where(row == 0, jnp.broadcast_to(vprev, rolled.shape), rolled)

    out_ref[0, 0] = w + jnp.dot(x, ws_ref[...],
                                preferred_element_type=jnp.float32)
    out_ref[0, 1] = m_ref[0, 0]
    out_ref[0, 2] = e


def kernel(input_data, Wq, Wk, Wv, Wo, Wself):
    B, C, N, D = input_data.shape
    R = 1000
    nb = N // R

    tail = input_data[:, :, N - 1, :]
    wxo = Wv[:D] @ Wo
    weo = Wv[D:] @ Wo

    return pl.pallas_call(
        _xattn_kernel,
        grid=(B, nb),
        in_specs=[
            pl.BlockSpec((1, 1, R, D), lambda b, j: (b, 0, j, 0)),
            pl.BlockSpec((1, 1, R, D), lambda b, j: (b, 1, j, 0)),
            pl.BlockSpec((1, 1, R, D), lambda b, j: (b, 2, j, 0)),
            pl.BlockSpec((1, C, D), lambda b, j: (b, 0, 0)),
            pl.BlockSpec((D, D), lambda b, j: (0, 0)),
            pl.BlockSpec((D, D), lambda b, j: (0, 0)),
            pl.BlockSpec((D, D), lambda b, j: (0, 0)),
        ],
        out_specs=pl.BlockSpec((1, C, R, D), lambda b, j: (b, 0, j, 0)),
        out_shape=jax.ShapeDtypeStruct((B, C, N, D), jnp.float32),
        scratch_shapes=[pltpu.VMEM((8, 128), jnp.float32)],
    )(input_data, input_data, input_data, tail, wxo, weo, Wself)


# speedup vs baseline: 1.1552x; 1.1552x over previous
import jax
import jax.numpy as jnp
from jax.experimental import pallas as pl
from jax.experimental.pallas import tpu as pltpu


def _xattn_kernel(x_ref, m_ref, e_ref, tail_ref, wxo_ref, weo_ref, ws_ref,
                  out_ref, carry_ref):
    j = pl.program_id(1)
    x = x_ref[0, 0]
    e = e_ref[0, 0]
    wxo = wxo_ref[...]
    weo = weo_ref[...]

    vexp = (jnp.dot(x, wxo, preferred_element_type=jnp.float32)
            + jnp.dot(e, weo, preferred_element_type=jnp.float32))

    tx = tail_ref[0, 0:1, :]
    te = tail_ref[0, 2:3, :]
    tail_vexp = (jnp.dot(tx, wxo, preferred_element_type=jnp.float32)
                 + jnp.dot(te, weo, preferred_element_type=jnp.float32))
    vprev = jnp.where(j == 0, tail_vexp, carry_ref[0:1, 0:3])
    carry_ref[0:1, 0:3] = vexp[-1:, :]

    rolled = pltpu.roll(vexp, 1, 0)
    row = jax.lax.broadcasted_iota(jnp.int32, rolled.shape, 0)
    w = jnp.where(row == 0, jnp.broadcast_to(vprev, rolled.shape), rolled)

    out_ref[0, 0] = w + jnp.dot(x, ws_ref[...],
                                preferred_element_type=jnp.float32)
    out_ref[0, 1] = m_ref[0, 0]
    out_ref[0, 2] = e


def kernel(input_data, Wq, Wk, Wv, Wo, Wself):
    B, C, N, D = input_data.shape
    R = 5000
    nb = N // R

    tail = input_data[:, :, N - 1, :]
    wxo = Wv[:D] @ Wo
    weo = Wv[D:] @ Wo

    return pl.pallas_call(
        _xattn_kernel,
        grid=(B, nb),
        in_specs=[
            pl.BlockSpec((1, 1, R, D), lambda b, j: (b, 0, j, 0)),
            pl.BlockSpec((1, 1, R, D), lambda b, j: (b, 1, j, 0)),
            pl.BlockSpec((1, 1, R, D), lambda b, j: (b, 2, j, 0)),
            pl.BlockSpec((1, C, D), lambda b, j: (b, 0, 0)),
            pl.BlockSpec((D, D), lambda b, j: (0, 0)),
            pl.BlockSpec((D, D), lambda b, j: (0, 0)),
            pl.BlockSpec((D, D), lambda b, j: (0, 0)),
        ],
        out_specs=pl.BlockSpec((1, C, R, D), lambda b, j: (b, 0, j, 0)),
        out_shape=jax.ShapeDtypeStruct((B, C, N, D), jnp.float32),
        scratch_shapes=[pltpu.VMEM((8, 128), jnp.float32)],
    )(input_data, input_data, input_data, tail, wxo, weo, Wself)


# speedup vs baseline: 16.9160x; 14.6433x over previous
import jax
import jax.numpy as jnp
from jax.experimental import pallas as pl
from jax.experimental.pallas import tpu as pltpu


def _xattn_kernel(x_ref, a_ref, av_ref, p_ref, out_ref):
    x = x_ref[...]
    vexp = jnp.dot(av_ref[...], x, preferred_element_type=jnp.float32)
    vs = pltpu.roll(vexp, 1, 1)
    out_ref[...] = (jnp.dot(a_ref[...], x, preferred_element_type=jnp.float32)
                    + jnp.dot(p_ref[...], vs,
                              preferred_element_type=jnp.float32))


def kernel(input_data, Wq, Wk, Wv, Wo, Wself):
    B, C, N, D = input_data.shape
    F = C * D * B
    eye_b = jnp.eye(B, dtype=jnp.float32)
    eye_db = jnp.eye(D * B, dtype=jnp.float32)

    a_mat = jnp.zeros((F, F), dtype=jnp.float32)
    a_mat = a_mat.at[0:D * B, 0:D * B].set(jnp.kron(Wself.T, eye_b))
    a_mat = a_mat.at[D * B:2 * D * B, D * B:2 * D * B].set(eye_db)
    a_mat = a_mat.at[2 * D * B:, 2 * D * B:].set(eye_db)

    av_mat = jnp.zeros((D * B, F), dtype=jnp.float32)
    av_mat = av_mat.at[:, 0:D * B].set(jnp.kron(jnp.outer(Wo[0], Wv[:D, 0]),
                                                eye_b))
    av_mat = av_mat.at[:, 2 * D * B:].set(jnp.kron(jnp.outer(Wo[0], Wv[D:, 0]),
                                                   eye_b))

    p_mat = jnp.zeros((F, D * B), dtype=jnp.float32)
    p_mat = p_mat.at[0:D * B, :].set(eye_db)

    x36 = input_data.transpose(1, 3, 0, 2).reshape(F, N)
    y36 = pl.pallas_call(
        _xattn_kernel,
        in_specs=[
            pl.BlockSpec((F, N), lambda: (0, 0)),
            pl.BlockSpec((F, F), lambda: (0, 0)),
            pl.BlockSpec((D * B, F), lambda: (0, 0)),
            pl.BlockSpec((F, D * B), lambda: (0, 0)),
        ],
        out_specs=pl.BlockSpec((F, N), lambda: (0, 0)),
        out_shape=jax.ShapeDtypeStruct((F, N), jnp.float32),
    )(x36, a_mat, av_mat, p_mat)
    return y36.reshape(C, D, B, N).transpose(2, 0, 3, 1)
